# baseline (speedup 1.0000x reference)
import functools

import jax
import jax.numpy as jnp
from jax import lax
from jax.experimental import pallas as pl
from jax.experimental.pallas import tpu as pltpu
from jax.experimental.pallas import tpu_sc as plsc

_VOCAB = 100000
_DIM = 150
_MAX_NORM = 1.0
_BATCH = 1024
_CTX = 20
_N = _BATCH * _CTX
_NC, _NS = 2, 16
_NW = _NC * _NS
_BPW = _N // _NW
_CHUNK = 128
_DIMP = 256
_PBLK = 4096
_VBLK = 4096


def _prep(table_t):
    def body(a_ref, out_ref):
        a = a_ref[...]
        e = (lax.broadcasted_iota(jnp.int32, (_DIM, _DIMP), 0) ==
             lax.broadcasted_iota(jnp.int32, (_DIM, _DIMP), 1)
             ).astype(jnp.float32)
        t = lax.dot_general(a, e, (((0,), (0,)), ((), ())),
                            preferred_element_type=jnp.float32)
        ss = jnp.sum(t * t, axis=1, keepdims=True)
        norm = jnp.sqrt(ss)
        scale = jnp.minimum(1.0, _MAX_NORM / jnp.maximum(norm, 1e-7))
        out_ref[...] = t * (scale * (1.0 / _CTX))

    return pl.pallas_call(
        body,
        grid=(pl.cdiv(_VOCAB, _PBLK),),
        in_specs=[pl.BlockSpec((_DIM, _PBLK), lambda i: (0, i))],
        out_specs=pl.BlockSpec((_PBLK, _DIMP), lambda i: (i, 0)),
        out_shape=jax.ShapeDtypeStruct((_VOCAB, _DIMP), jnp.float32),
        compiler_params=pltpu.CompilerParams(
            dimension_semantics=("parallel",)),
    )(table_t)


_BPC = 4
_CROWS = _BPC * _CTX
_BW = _BATCH // _NW
_NCHUNK = _BW // _BPC
_NLANE = 160 // 16


def _sc_gather_pool(flat_idx, table):
    mesh = plsc.VectorSubcoreMesh(core_axis_name="c", subcore_axis_name="s")

    @functools.partial(
        pl.kernel,
        out_type=jax.ShapeDtypeStruct((_BATCH, _DIMP), jnp.float32),
        mesh=mesh,
        scratch_types=[
            pltpu.VMEM((_BPW,), jnp.int32),
            pltpu.VMEM((_CROWS, _DIMP), jnp.float32),
            pltpu.VMEM((_CROWS, _DIMP), jnp.float32),
            pltpu.VMEM((_BW, _DIMP), jnp.float32),
            pltpu.SemaphoreType.DMA,
            pltpu.SemaphoreType.DMA,
        ],
    )
    def gather_kernel(idx_hbm, table_hbm, x_hbm, idx_v, buf0, buf1, xacc,
                      s0, s1):
        wid = lax.axis_index("s") * _NC + lax.axis_index("c")
        base = wid * _BPW
        pltpu.sync_copy(idx_hbm.at[pl.ds(base, _BPW)], idx_v)
        bufs = (buf0, buf1)
        sems = (s0, s1)

        def fire(j):
            return pltpu.async_copy(
                table_hbm.at[idx_v.at[pl.ds(j * _CROWS, _CROWS)]],
                bufs[j % 2], sems[j % 2])

        def pool_chunk(j):
            buf = bufs[j % 2]
            for q in range(_BPC):
                row0 = q * _CTX

                def body(r, acc):
                    return tuple(
                        acc[c] + buf[row0 + r, pl.ds(c * 16, 16)]
                        for c in range(_NLANE))

                acc0 = tuple(
                    buf[row0, pl.ds(c * 16, 16)] for c in range(_NLANE))
                acc = lax.fori_loop(1, _CTX, body, acc0)
                for c in range(_NLANE):
                    xacc[j * _BPC + q, pl.ds(c * 16, 16)] = acc[c]

        copies = [fire(0), fire(1)]
        for j in range(2, _NCHUNK):
            copies[j - 2].wait()
            pool_chunk(j - 2)
            copies.append(fire(j))
        for j in range(_NCHUNK, _NCHUNK + 2):
            copies[j - 2].wait()
            pool_chunk(j - 2)
        pltpu.sync_copy(xacc, x_hbm.at[pl.ds(wid * _BW, _BW)])

    return gather_kernel(flat_idx, table)


def _linear(x, w_t, b2):
    def body(x_ref, w_ref, b_ref, out_ref):
        out_ref[...] = lax.dot_general(
            w_ref[...], x_ref[...],
            (((0,), (1,)), ((), ())),
            preferred_element_type=jnp.float32,
        ) + b_ref[...]

    return pl.pallas_call(
        body,
        grid=(pl.cdiv(_VOCAB, _VBLK),),
        in_specs=[
            pl.BlockSpec((_BATCH, _DIM), lambda i: (0, 0)),
            pl.BlockSpec((_DIM, _VBLK), lambda i: (0, i)),
            pl.BlockSpec((_VBLK, 1), lambda i: (i, 0)),
        ],
        out_specs=pl.BlockSpec((_VBLK, _BATCH), lambda i: (i, 0)),
        out_shape=jax.ShapeDtypeStruct((_VOCAB, _BATCH), jnp.float32),
        compiler_params=pltpu.CompilerParams(
            dimension_semantics=("parallel",)),
    )(x, w_t, b2)


def kernel(input_features, emb_table, W, b):
    flat_idx = input_features.reshape(-1).astype(jnp.int32)
    table_p = _prep(emb_table.T)
    x = _sc_gather_pool(flat_idx, table_p)[:, :_DIM]
    out_t = _linear(x, W.T, b.reshape(_VOCAB, 1))
    return out_t.T


# speedup vs baseline: 1.0206x; 1.0206x over previous
import functools

import jax
import jax.numpy as jnp
from jax import lax
from jax.experimental import pallas as pl
from jax.experimental.pallas import tpu as pltpu
from jax.experimental.pallas import tpu_sc as plsc

_VOCAB = 100000
_DIM = 150
_MAX_NORM = 1.0
_BATCH = 1024
_CTX = 20
_N = _BATCH * _CTX
_NC, _NS = 2, 16
_NW = _NC * _NS
_BPW = _N // _NW
_CHUNK = 128
_DIMP = 256
_PBLK = 8192
_VBLK = 4096


def _prep(table_t):
    def body(a_ref, out_ref):
        a = a_ref[...]
        e = (lax.broadcasted_iota(jnp.int32, (_DIM, _DIMP), 0) ==
             lax.broadcasted_iota(jnp.int32, (_DIM, _DIMP), 1)
             ).astype(jnp.float32)
        t = lax.dot_general(a, e, (((0,), (0,)), ((), ())),
                            preferred_element_type=jnp.float32)
        ss = jnp.sum(t * t, axis=1, keepdims=True)
        norm = jnp.sqrt(ss)
        scale = jnp.minimum(1.0, _MAX_NORM / jnp.maximum(norm, 1e-7))
        out_ref[...] = t * (scale * (1.0 / _CTX))

    return pl.pallas_call(
        body,
        grid=(pl.cdiv(_VOCAB, _PBLK),),
        in_specs=[pl.BlockSpec((_DIM, _PBLK), lambda i: (0, i))],
        out_specs=pl.BlockSpec((_PBLK, _DIMP), lambda i: (i, 0)),
        out_shape=jax.ShapeDtypeStruct((_VOCAB, _DIMP), jnp.float32),
        compiler_params=pltpu.CompilerParams(
            dimension_semantics=("parallel",)),
    )(table_t)


_BPC = 4
_CROWS = _BPC * _CTX
_BW = _BATCH // _NW
_NCHUNK = _BW // _BPC
_NLANE = 160 // 16


def _sc_gather_pool(flat_idx, table):
    mesh = plsc.VectorSubcoreMesh(core_axis_name="c", subcore_axis_name="s")

    @functools.partial(
        pl.kernel,
        out_type=jax.ShapeDtypeStruct((_BATCH, _DIMP), jnp.float32),
        mesh=mesh,
        scratch_types=[
            pltpu.VMEM((_BPW,), jnp.int32),
            pltpu.VMEM((_CROWS, _DIMP), jnp.float32),
            pltpu.VMEM((_CROWS, _DIMP), jnp.float32),
            pltpu.VMEM((_BW, _DIMP), jnp.float32),
            pltpu.SemaphoreType.DMA,
            pltpu.SemaphoreType.DMA,
        ],
    )
    def gather_kernel(idx_hbm, table_hbm, x_hbm, idx_v, buf0, buf1, xacc,
                      s0, s1):
        wid = lax.axis_index("s") * _NC + lax.axis_index("c")
        base = wid * _BPW
        pltpu.sync_copy(idx_hbm.at[pl.ds(base, _BPW)], idx_v)
        bufs = (buf0, buf1)
        sems = (s0, s1)

        def fire(j):
            return pltpu.async_copy(
                table_hbm.at[idx_v.at[pl.ds(j * _CROWS, _CROWS)]],
                bufs[j % 2], sems[j % 2])

        def pool_chunk(j):
            buf = bufs[j % 2]
            for q in range(_BPC):
                row0 = q * _CTX

                def body(r, acc):
                    return tuple(
                        acc[c] + buf[row0 + r, pl.ds(c * 16, 16)]
                        for c in range(_NLANE))

                acc0 = tuple(
                    buf[row0, pl.ds(c * 16, 16)] for c in range(_NLANE))
                acc = lax.fori_loop(1, _CTX, body, acc0)
                for c in range(_NLANE):
                    xacc[j * _BPC + q, pl.ds(c * 16, 16)] = acc[c]

        copies = [fire(0), fire(1)]
        for j in range(2, _NCHUNK):
            copies[j - 2].wait()
            pool_chunk(j - 2)
            copies.append(fire(j))
        for j in range(_NCHUNK, _NCHUNK + 2):
            copies[j - 2].wait()
            pool_chunk(j - 2)
        pltpu.sync_copy(xacc, x_hbm.at[pl.ds(wid * _BW, _BW)])

    return gather_kernel(flat_idx, table)


def _linear(x, w_t, b2):
    def body(x_ref, w_ref, b_ref, out_ref):
        out_ref[...] = lax.dot_general(
            w_ref[...], x_ref[...],
            (((0,), (1,)), ((), ())),
            preferred_element_type=jnp.float32,
        ) + b_ref[...]

    return pl.pallas_call(
        body,
        grid=(pl.cdiv(_VOCAB, _VBLK),),
        in_specs=[
            pl.BlockSpec((_BATCH, _DIM), lambda i: (0, 0)),
            pl.BlockSpec((_DIM, _VBLK), lambda i: (0, i)),
            pl.BlockSpec((_VBLK, 1), lambda i: (i, 0)),
        ],
        out_specs=pl.BlockSpec((_VBLK, _BATCH), lambda i: (i, 0)),
        out_shape=jax.ShapeDtypeStruct((_VOCAB, _BATCH), jnp.float32),
        compiler_params=pltpu.CompilerParams(
            dimension_semantics=("parallel",)),
    )(x, w_t, b2)


def kernel(input_features, emb_table, W, b):
    flat_idx = input_features.reshape(-1).astype(jnp.int32)
    table_p = _prep(emb_table.T)
    x = _sc_gather_pool(flat_idx, table_p)[:, :_DIM]
    out_t = _linear(x, W.T, b.reshape(_VOCAB, 1))
    return out_t.T
